# baseline (speedup 1.0000x reference)
import jax
import jax.numpy as jnp
from jax.experimental import pallas as pl


def _expand_scale_kernel(w_ref, t_ref, o_ref):
    w = jnp.maximum(w_ref[...], 0.0) + 1e-9
    lane = jax.lax.broadcasted_iota(jnp.int32, (64, 128), 1)
    src = jax.lax.broadcasted_iota(jnp.int32, (64, 128), 0)
    is_ch1 = (lane // 16) % 2 == 1
    sel = ((lane // 32) * 16 + (lane % 16)) == src
    scatter = jnp.where(is_ch1 & sel, 1.0, 0.0)
    m = jax.lax.dot(w, scatter, precision=jax.lax.Precision.HIGHEST,
                    preferred_element_type=jnp.float32)
    lane_row = jax.lax.broadcasted_iota(jnp.int32, (1, 128), 1)
    ones_ch0 = jnp.where((lane_row // 16) % 2 == 0, 1.0, 0.0)
    o_ref[...] = t_ref[...] * (m + ones_ch0)


def _block_rows(rows: int) -> int:
    for cand in range(min(rows, 2048), 0, -1):
        if rows % cand == 0:
            return cand
    return rows


def kernel(t, weights):
    v = t.shape[0]
    rows = v // 4
    t2 = t.reshape(rows, 128)
    w2 = weights.reshape(rows, 64)
    b = _block_rows(rows)
    out = pl.pallas_call(
        _expand_scale_kernel,
        grid=(rows // b,),
        in_specs=[
            pl.BlockSpec((b, 64), lambda i: (i, 0)),
            pl.BlockSpec((b, 128), lambda i: (i, 0)),
        ],
        out_specs=pl.BlockSpec((b, 128), lambda i: (i, 0)),
        out_shape=jax.ShapeDtypeStruct((rows, 128), jnp.float32),
    )(w2, t2)
    return out.reshape(v, 2, 16)


# speedup vs baseline: 1.0079x; 1.0079x over previous
import numpy as np
import jax
import jax.numpy as jnp
from jax.experimental import pallas as pl

_lane = np.arange(128)
_src = np.arange(64)[:, None]
_SCATTER = (((_lane // 16) % 2 == 1)
            & (((_lane // 32) * 16 + (_lane % 16)) == _src)).astype(np.float32)
_CH0_ONES = (((_lane // 16) % 2 == 0).astype(np.float32))[None, :]


def _expand_scale_kernel(w_ref, s_ref, c_ref, t_ref, o_ref):
    w = jnp.maximum(w_ref[...], 0.0) + 1e-9
    m = jax.lax.dot(w.astype(jnp.bfloat16), s_ref[...],
                    preferred_element_type=jnp.float32)
    o_ref[...] = t_ref[...] * (m + c_ref[...])


def _block_rows(rows: int) -> int:
    for cand in range(min(rows, 2048), 0, -1):
        if rows % cand == 0:
            return cand
    return rows


def kernel(t, weights):
    v = t.shape[0]
    rows = v // 4
    t2 = t.reshape(rows, 128)
    w2 = weights.reshape(rows, 64)
    b = _block_rows(rows)
    out = pl.pallas_call(
        _expand_scale_kernel,
        grid=(rows // b,),
        in_specs=[
            pl.BlockSpec((b, 64), lambda i: (i, 0)),
            pl.BlockSpec((64, 128), lambda i: (0, 0)),
            pl.BlockSpec((1, 128), lambda i: (0, 0)),
            pl.BlockSpec((b, 128), lambda i: (i, 0)),
        ],
        out_specs=pl.BlockSpec((b, 128), lambda i: (i, 0)),
        out_shape=jax.ShapeDtypeStruct((rows, 128), jnp.float32),
    )(w2, jnp.asarray(_SCATTER, dtype=jnp.bfloat16), jnp.asarray(_CH0_ONES), t2)
    return out.reshape(v, 2, 16)


# speedup vs baseline: 62.4248x; 61.9325x over previous
import jax
import jax.numpy as jnp
from jax.experimental import pallas as pl

_LANE_BLOCK = 65536


def _scale_kernel(w_ref, t_ref, o_ref):
    o_ref[0] = t_ref[0]
    pw = jnp.maximum(w_ref[...], 0.0) + 1e-9
    o_ref[1] = t_ref[1] * pw


def kernel(t, weights):
    v, _, width = t.shape
    tt = jnp.transpose(t, (1, 2, 0))
    wt = jnp.transpose(weights, (1, 0))
    c = min(_LANE_BLOCK, v)
    g = -(-v // c)
    out = pl.pallas_call(
        _scale_kernel,
        grid=(g,),
        in_specs=[
            pl.BlockSpec((width, c), lambda i: (0, i)),
            pl.BlockSpec((2, width, c), lambda i: (0, 0, i)),
        ],
        out_specs=pl.BlockSpec((2, width, c), lambda i: (0, 0, i)),
        out_shape=jax.ShapeDtypeStruct((2, width, v), jnp.float32),
    )(wt, tt)
    return jnp.transpose(out, (2, 0, 1))
